# baseline (speedup 1.0000x reference)
import functools

import jax
import jax.numpy as jnp
from jax import lax
from jax.experimental import pallas as pl
from jax.experimental.pallas import tpu as pltpu
from jax.experimental.pallas import tpu_sc as plsc

N = 10000
E = 320000
D = 128
H = 4
DH = 32
HOPS = 2
ALPHA = 0.15
C = 40

NPAD = 10240
EPAD = 327680
NTILES = 32
PER_TILE = EPAD // NTILES
K = 128
NCHUNK = PER_TILE // K
KA = 512
NCHA = PER_TILE // KA
ROWS_PER_TILE = NPAD // 16

f32 = jnp.float32
i32 = jnp.int32


def _wid(c, s):
    return s * 2 + c


def _proj_body(h_ref, w_ref, p_ref, hp_ref, elr_ref):
    hp = jnp.dot(h_ref[...], w_ref[...], preferred_element_type=f32)
    hp_ref[...] = hp
    elr_ref[...] = lax.dot_general(
        p_ref[...], hp, (((0,), (1,)), ((), ())), preferred_element_type=f32)


def _proj(h, w, p, br=1024):
    grid = (NPAD // br,)
    return pl.pallas_call(
        _proj_body,
        grid=grid,
        in_specs=[
            pl.BlockSpec((br, D), lambda i: (i, 0)),
            pl.BlockSpec((D, D), lambda i: (0, 0)),
            pl.BlockSpec((D, 2 * H), lambda i: (0, 0)),
        ],
        out_specs=[
            pl.BlockSpec((br, D), lambda i: (i, 0)),
            pl.BlockSpec((2 * H, br), lambda i: (0, i)),
        ],
        out_shape=[
            jax.ShapeDtypeStruct((NPAD, D), f32),
            jax.ShapeDtypeStruct((2 * H, NPAD), f32),
        ],
    )(h, w, p)


def _recip_body(d_ref, r_ref):
    s = jnp.sum(d_ref[...], axis=0, keepdims=True)
    r_ref[...] = 1.0 / (s + 1e-9)


def _recip(dparts32):
    return pl.pallas_call(
        _recip_body,
        out_shape=jax.ShapeDtypeStruct((1, NPAD * H), f32),
    )(dparts32)


def _combine_mid_body(pa_ref, hp_ref, o_ref):
    agg = pa_ref[0] + pa_ref[1]
    o_ref[...] = (1.0 - ALPHA) * agg + ALPHA * hp_ref[...]


def _combine_mid(parts, hp, br=1024):
    parts = parts.reshape(2, NPAD, D)
    return pl.pallas_call(
        _combine_mid_body,
        grid=(NPAD // br,),
        in_specs=[
            pl.BlockSpec((2, br, D), lambda i: (0, i, 0)),
            pl.BlockSpec((br, D), lambda i: (i, 0)),
        ],
        out_specs=pl.BlockSpec((br, D), lambda i: (i, 0)),
        out_shape=jax.ShapeDtypeStruct((NPAD, D), f32),
    )(parts, hp)


def _combine_end_body(pa_ref, hp_ref, hin_ref, o_ref):
    agg = pa_ref[0] + pa_ref[1]
    z = (1.0 - ALPHA) * agg + ALPHA * hp_ref[...] + hin_ref[...]
    o_ref[...] = jnp.where(z > 0, z, jnp.exp(jnp.minimum(z, 0.0)) - 1.0)


def _combine_end(parts, hp, hin, br=1024):
    parts = parts.reshape(2, NPAD, D)
    return pl.pallas_call(
        _combine_end_body,
        grid=(NPAD // br,),
        in_specs=[
            pl.BlockSpec((2, br, D), lambda i: (0, i, 0)),
            pl.BlockSpec((br, D), lambda i: (i, 0)),
            pl.BlockSpec((br, D), lambda i: (i, 0)),
        ],
        out_specs=pl.BlockSpec((br, D), lambda i: (i, 0)),
        out_shape=jax.ShapeDtypeStruct((NPAD, D), f32),
    )(parts, hp, hin)


def _classify_body(pa_ref, hp_ref, hin_ref, wc_ref, bc_ref, o_ref):
    agg = pa_ref[0] + pa_ref[1]
    z = (1.0 - ALPHA) * agg + ALPHA * hp_ref[...] + hin_ref[...]
    h = jnp.where(z > 0, z, jnp.exp(jnp.minimum(z, 0.0)) - 1.0)
    o_ref[...] = (jnp.dot(h, wc_ref[...], preferred_element_type=f32)
                  + bc_ref[...])


def _classify(parts, hp, hin, wc_pad, bc_pad, br=1024):
    parts = parts.reshape(2, NPAD, D)
    return pl.pallas_call(
        _classify_body,
        grid=(NPAD // br,),
        in_specs=[
            pl.BlockSpec((2, br, D), lambda i: (0, i, 0)),
            pl.BlockSpec((br, D), lambda i: (i, 0)),
            pl.BlockSpec((br, D), lambda i: (i, 0)),
            pl.BlockSpec((D, D), lambda i: (0, 0)),
            pl.BlockSpec((1, D), lambda i: (0, 0)),
        ],
        out_specs=pl.BlockSpec((br, D), lambda i: (i, 0)),
        out_shape=jax.ShapeDtypeStruct((NPAD, D), f32),
    )(parts, hp, hin, wc_pad, bc_pad)


_MESH = plsc.VectorSubcoreMesh(core_axis_name="c", subcore_axis_name="s")


def _attn_body(elr_hbm, srcp, dstp, zeros4, ex_hbm, dpart_hbm,
               elr_v, denom_v, sidx, didx, exb):
    c = lax.axis_index("c")
    s = lax.axis_index("s")
    w = _wid(c, s)
    iota = lax.iota(i32, 16)

    pltpu.sync_copy(elr_hbm, elr_v)
    pltpu.sync_copy(zeros4, denom_v)

    def chunk(ci, _):
        ebase = w * PER_TILE + ci * KA
        pltpu.sync_copy(srcp.at[pl.ds(ebase, KA)], sidx)
        pltpu.sync_copy(dstp.at[pl.ds(ebase, KA)], didx)
        for g in range(KA // 16):
            sv = sidx[pl.ds(g * 16, 16)]
            dv = didx[pl.ds(g * 16, 16)]
            eid = ebase + g * 16 + iota
            valid = eid < E
            pos = (g * 16 + iota) * H
            for h in range(H):
                el = plsc.load_gather(elr_v, [sv + h * NPAD])
                er = plsc.load_gather(elr_v, [dv + (H + h) * NPAD])
                e = el + er
                e = jnp.where(e >= 0, e, 0.2 * e)
                ex = jnp.where(valid, jnp.exp(e), 0.0)
                plsc.store_scatter(exb, [pos + h], ex)
                plsc.addupdate_scatter(denom_v, [dv * H + h], ex)
        pltpu.sync_copy(exb, ex_hbm.at[pl.ds(ebase * H, KA * H)])
        return _

    lax.fori_loop(0, NCHA, chunk, None)
    pltpu.sync_copy(denom_v, dpart_hbm.at[pl.ds(w * NPAD * H, NPAD * H)])


def _attn(elr_flat, srcp, dstp, zeros4):
    return pl.kernel(
        _attn_body,
        out_type=[
            jax.ShapeDtypeStruct((EPAD * H,), f32),
            jax.ShapeDtypeStruct((NTILES * NPAD * H,), f32),
        ],
        mesh=_MESH,
        compiler_params=pltpu.CompilerParams(needs_layout_passes=False),
        scratch_types=[
            pltpu.VMEM((2 * H * NPAD,), f32),
            pltpu.VMEM((NPAD * H,), f32),
            pltpu.VMEM((KA,), i32),
            pltpu.VMEM((KA,), i32),
            pltpu.VMEM((KA * H,), f32),
        ],
    )(elr_flat, srcp, dstp, zeros4)


def _attn_scale_body(recip_hbm, ex_hbm, dstp, attn_hbm,
                     recip_v, didx, exb, attnb):
    c = lax.axis_index("c")
    s = lax.axis_index("s")
    w = _wid(c, s)
    iota = lax.iota(i32, 16)

    pltpu.sync_copy(recip_hbm, recip_v)

    def chunk(ci, _):
        ebase = w * PER_TILE + ci * KA
        pltpu.sync_copy(dstp.at[pl.ds(ebase, KA)], didx)
        pltpu.sync_copy(ex_hbm.at[pl.ds(ebase * H, KA * H)], exb)
        for g in range(KA // 16):
            dv = didx[pl.ds(g * 16, 16)]
            pos = (g * 16 + iota) * H
            for h in range(H):
                rv = plsc.load_gather(recip_v, [dv * H + h])
                exv = plsc.load_gather(exb, [pos + h])
                plsc.store_scatter(attnb, [pos + h], rv * exv)
        pltpu.sync_copy(attnb, attn_hbm.at[pl.ds(ebase * H, KA * H)])
        return _

    lax.fori_loop(0, NCHA, chunk, None)


def _attn_scale(recip_flat, ex, dstp):
    return pl.kernel(
        _attn_scale_body,
        out_type=jax.ShapeDtypeStruct((EPAD * H,), f32),
        mesh=_MESH,
        compiler_params=pltpu.CompilerParams(needs_layout_passes=False),
        scratch_types=[
            pltpu.VMEM((NPAD * H,), f32),
            pltpu.VMEM((KA,), i32),
            pltpu.VMEM((KA * H,), f32),
            pltpu.VMEM((KA * H,), f32),
        ],
    )(recip_flat, ex, dstp)


def _hop_body(feat_hbm, attn_hbm, srcp, dstp, zeros128, apart_hbm,
              rows0, sidx0, didx0, scaleb0, rows1, sidx1, didx1, scaleb1,
              agg_sh, g0, g1):
    c = lax.axis_index("c")
    s = lax.axis_index("s")
    w = _wid(c, s)

    pltpu.sync_copy(zeros128.at[pl.ds(s * ROWS_PER_TILE, ROWS_PER_TILE)],
                    agg_sh.at[pl.ds(s * ROWS_PER_TILE, ROWS_PER_TILE)])
    plsc.subcore_barrier()

    def issue(ci, sidx, didx, scaleb, rows, gsem):
        ebase = w * PER_TILE + ci * K
        pltpu.sync_copy(srcp.at[pl.ds(ebase, K)], sidx)
        pltpu.sync_copy(dstp.at[pl.ds(ebase, K)], didx)
        pltpu.sync_copy(attn_hbm.at[pl.ds(ebase * H, K * H)], scaleb)
        pltpu.async_copy(feat_hbm.at[sidx], rows, gsem)

    def process(sidx, didx, scaleb, rows, gsem):
        pltpu.make_async_copy(feat_hbm.at[sidx], rows, gsem).wait()

        def edge4(e4, _):
            scv16 = scaleb[pl.ds(e4 * 16, 16)]
            for q in range(4):
                e = e4 * 4 + q
                for h in range(H):
                    lane = jnp.full((16,), q * H + h, i32)
                    scv = scv16.at[lane].get(mode="promise_in_bounds")
                    for j in range(DH // 16):
                        off = h * DH + j * 16
                        rows[e, pl.ds(off, 16)] = rows[e, pl.ds(off, 16)] * scv
            return _

        lax.fori_loop(0, K // 4, edge4, None)
        pltpu.sync_copy(rows, agg_sh.at[didx], add=True)

    NPAIR = NCHUNK // 2
    issue(0, sidx0, didx0, scaleb0, rows0, g0)

    def pair(cj, _):
        issue(2 * cj + 1, sidx1, didx1, scaleb1, rows1, g1)
        process(sidx0, didx0, scaleb0, rows0, g0)

        @pl.when(cj < NPAIR - 1)
        def _issue_next():
            issue(2 * cj + 2, sidx0, didx0, scaleb0, rows0, g0)

        process(sidx1, didx1, scaleb1, rows1, g1)
        return _

    lax.fori_loop(0, NPAIR, pair, None)
    plsc.subcore_barrier()
    r0 = s * ROWS_PER_TILE
    pltpu.sync_copy(agg_sh.at[pl.ds(r0, ROWS_PER_TILE)],
                    apart_hbm.at[pl.ds(c * NPAD + r0, ROWS_PER_TILE)])


def _hop(feat, attn, srcp, dstp, zeros128):
    return pl.kernel(
        _hop_body,
        out_type=jax.ShapeDtypeStruct((2 * NPAD, D), f32),
        mesh=_MESH,
        compiler_params=pltpu.CompilerParams(needs_layout_passes=False),
        scratch_types=[
            pltpu.VMEM((K, D), f32),
            pltpu.VMEM((K,), i32),
            pltpu.VMEM((K,), i32),
            pltpu.VMEM((K * H,), f32),
            pltpu.VMEM((K, D), f32),
            pltpu.VMEM((K,), i32),
            pltpu.VMEM((K,), i32),
            pltpu.VMEM((K * H,), f32),
            pltpu.VMEM_SHARED((NPAD, D), f32),
            pltpu.SemaphoreType.DMA,
            pltpu.SemaphoreType.DMA,
        ],
    )(feat, attn, srcp, dstp, zeros128)


def _layer(h_in, w, p, srcp, dstp, zeros4, zeros128):
    hp, elr = _proj(h_in, w, p)
    ex, dparts = _attn(elr.reshape(-1), srcp, dstp, zeros4)
    recip = _recip(dparts.reshape(NTILES, NPAD * H)).reshape(-1)
    attn = _attn_scale(recip, ex, dstp)
    feat = hp
    parts = None
    for k in range(HOPS):
        parts = _hop(feat, attn, srcp, dstp, zeros128)
        if k < HOPS - 1:
            feat = _combine_mid(parts, hp)
    return parts, hp


def kernel(x, edge_index, W1, al1, ar1, W2, al2, ar2, Wc, bc):
    xp = jnp.zeros((NPAD, D), f32).at[:N].set(x)
    srcp = jnp.zeros((EPAD,), i32).at[:E].set(edge_index[0])
    dstp = jnp.zeros((EPAD,), i32).at[:E].set(edge_index[1])
    mask = jnp.repeat(jnp.eye(H, dtype=f32), DH, axis=0)
    p1 = jnp.concatenate([mask * al1.reshape(-1)[:, None],
                          mask * ar1.reshape(-1)[:, None]], axis=1)
    p2 = jnp.concatenate([mask * al2.reshape(-1)[:, None],
                          mask * ar2.reshape(-1)[:, None]], axis=1)
    wc_pad = jnp.zeros((D, D), f32).at[:, :C].set(Wc)
    bc_pad = jnp.zeros((1, D), f32).at[0, :C].set(bc)
    zeros4 = jnp.zeros((NPAD * H,), f32)
    zeros128 = jnp.zeros((NPAD, D), f32)

    parts1, hp1 = _layer(xp, W1, p1, srcp, dstp, zeros4, zeros128)
    h1 = _combine_end(parts1, hp1, xp)
    parts2, hp2 = _layer(h1, W2, p2, srcp, dstp, zeros4, zeros128)
    logits_pad = _classify(parts2, hp2, h1, wc_pad, bc_pad)
    return logits_pad[:N, :C]


# speedup vs baseline: 1.0009x; 1.0009x over previous
import jax
import jax.numpy as jnp
from jax import lax
from jax.experimental import pallas as pl
from jax.experimental.pallas import tpu as pltpu
from jax.experimental.pallas import tpu_sc as plsc

N = 10000
E = 320000
D = 128
H = 4
DH = 32
HOPS = 2
ALPHA = 0.15
C = 40

NPAD = 10240
EPAD = 327680
NTILES = 32
PER_TILE = EPAD // NTILES
K = 128
NCHUNK = PER_TILE // K
KA = 512
NCHA = PER_TILE // KA
ROWS_PER_TILE = NPAD // 16

f32 = jnp.float32
i32 = jnp.int32


def _wid(c, s):
    return s * 2 + c


def _proj_body(h_ref, w_ref, p_ref, hp_ref, elr_ref):
    hp = jnp.dot(h_ref[...], w_ref[...], preferred_element_type=f32)
    hp_ref[...] = hp
    elr_ref[...] = lax.dot_general(
        p_ref[...], hp, (((0,), (1,)), ((), ())), preferred_element_type=f32)


def _proj(h, w, p, br=1024):
    grid = (NPAD // br,)
    return pl.pallas_call(
        _proj_body,
        grid=grid,
        in_specs=[
            pl.BlockSpec((br, D), lambda i: (i, 0)),
            pl.BlockSpec((D, D), lambda i: (0, 0)),
            pl.BlockSpec((D, 2 * H), lambda i: (0, 0)),
        ],
        out_specs=[
            pl.BlockSpec((br, D), lambda i: (i, 0)),
            pl.BlockSpec((2 * H, br), lambda i: (0, i)),
        ],
        out_shape=[
            jax.ShapeDtypeStruct((NPAD, D), f32),
            jax.ShapeDtypeStruct((2 * H, NPAD), f32),
        ],
    )(h, w, p)


def _recip_body(d_ref, r_ref):
    s = jnp.sum(d_ref[...], axis=0, keepdims=True)
    r_ref[...] = 1.0 / (s + 1e-9)


def _recip(dparts32):
    return pl.pallas_call(
        _recip_body,
        out_shape=jax.ShapeDtypeStruct((1, NPAD * H), f32),
    )(dparts32)


def _combine_mid_body(pa_ref, hp_ref, o_ref):
    agg = pa_ref[0] + pa_ref[1]
    o_ref[...] = (1.0 - ALPHA) * agg + ALPHA * hp_ref[...]


def _combine_mid(parts, hp, br=1024):
    parts = parts.reshape(2, NPAD, D)
    return pl.pallas_call(
        _combine_mid_body,
        grid=(NPAD // br,),
        in_specs=[
            pl.BlockSpec((2, br, D), lambda i: (0, i, 0)),
            pl.BlockSpec((br, D), lambda i: (i, 0)),
        ],
        out_specs=pl.BlockSpec((br, D), lambda i: (i, 0)),
        out_shape=jax.ShapeDtypeStruct((NPAD, D), f32),
    )(parts, hp)


def _combine_end_body(pa_ref, hp_ref, hin_ref, o_ref):
    agg = pa_ref[0] + pa_ref[1]
    z = (1.0 - ALPHA) * agg + ALPHA * hp_ref[...] + hin_ref[...]
    o_ref[...] = jnp.where(z > 0, z, jnp.exp(jnp.minimum(z, 0.0)) - 1.0)


def _combine_end(parts, hp, hin, br=1024):
    parts = parts.reshape(2, NPAD, D)
    return pl.pallas_call(
        _combine_end_body,
        grid=(NPAD // br,),
        in_specs=[
            pl.BlockSpec((2, br, D), lambda i: (0, i, 0)),
            pl.BlockSpec((br, D), lambda i: (i, 0)),
            pl.BlockSpec((br, D), lambda i: (i, 0)),
        ],
        out_specs=pl.BlockSpec((br, D), lambda i: (i, 0)),
        out_shape=jax.ShapeDtypeStruct((NPAD, D), f32),
    )(parts, hp, hin)


def _classify_body(pa_ref, hp_ref, hin_ref, wc_ref, bc_ref, o_ref):
    agg = pa_ref[0] + pa_ref[1]
    z = (1.0 - ALPHA) * agg + ALPHA * hp_ref[...] + hin_ref[...]
    h = jnp.where(z > 0, z, jnp.exp(jnp.minimum(z, 0.0)) - 1.0)
    o_ref[...] = (jnp.dot(h, wc_ref[...], preferred_element_type=f32)
                  + bc_ref[...])


def _classify(parts, hp, hin, wc_pad, bc_pad, br=1024):
    parts = parts.reshape(2, NPAD, D)
    return pl.pallas_call(
        _classify_body,
        grid=(NPAD // br,),
        in_specs=[
            pl.BlockSpec((2, br, D), lambda i: (0, i, 0)),
            pl.BlockSpec((br, D), lambda i: (i, 0)),
            pl.BlockSpec((br, D), lambda i: (i, 0)),
            pl.BlockSpec((D, D), lambda i: (0, 0)),
            pl.BlockSpec((1, D), lambda i: (0, 0)),
        ],
        out_specs=pl.BlockSpec((br, D), lambda i: (i, 0)),
        out_shape=jax.ShapeDtypeStruct((NPAD, D), f32),
    )(parts, hp, hin, wc_pad, bc_pad)


_MESH = plsc.VectorSubcoreMesh(core_axis_name="c", subcore_axis_name="s")


def _attn_body(elr_hbm, srcp, dstp, zeros4, ex_hbm, dpart_hbm,
               elr_v, denom_v, sidx, didx, exb):
    c = lax.axis_index("c")
    s = lax.axis_index("s")
    w = _wid(c, s)
    iota = lax.iota(i32, 16)

    pltpu.sync_copy(elr_hbm, elr_v)
    pltpu.sync_copy(zeros4, denom_v)

    def chunk(ci, _):
        ebase = w * PER_TILE + ci * KA
        pltpu.sync_copy(srcp.at[pl.ds(ebase, KA)], sidx)
        pltpu.sync_copy(dstp.at[pl.ds(ebase, KA)], didx)
        for g in range(KA // 16):
            sv = sidx[pl.ds(g * 16, 16)]
            dv = didx[pl.ds(g * 16, 16)]
            eid = ebase + g * 16 + iota
            valid = eid < E
            pos = (g * 16 + iota) * H
            for h in range(H):
                el = plsc.load_gather(elr_v, [sv + h * NPAD])
                er = plsc.load_gather(elr_v, [dv + (H + h) * NPAD])
                e = el + er
                e = jnp.where(e >= 0, e, 0.2 * e)
                ex = jnp.where(valid, jnp.exp(e), 0.0)
                plsc.store_scatter(exb, [pos + h], ex)
                plsc.addupdate_scatter(denom_v, [dv * H + h], ex)
        pltpu.sync_copy(exb, ex_hbm.at[pl.ds(ebase * H, KA * H)])
        return _

    lax.fori_loop(0, NCHA, chunk, None)
    pltpu.sync_copy(denom_v, dpart_hbm.at[pl.ds(w * NPAD * H, NPAD * H)])


def _attn(elr_flat, srcp, dstp, zeros4):
    return pl.kernel(
        _attn_body,
        out_type=[
            jax.ShapeDtypeStruct((EPAD * H,), f32),
            jax.ShapeDtypeStruct((NTILES * NPAD * H,), f32),
        ],
        mesh=_MESH,
        compiler_params=pltpu.CompilerParams(needs_layout_passes=False),
        scratch_types=[
            pltpu.VMEM((2 * H * NPAD,), f32),
            pltpu.VMEM((NPAD * H,), f32),
            pltpu.VMEM((KA,), i32),
            pltpu.VMEM((KA,), i32),
            pltpu.VMEM((KA * H,), f32),
        ],
    )(elr_flat, srcp, dstp, zeros4)


def _attn_scale_body(recip_hbm, ex_hbm, dstp, attn_hbm,
                     recip_v, didx, exb, attnb):
    c = lax.axis_index("c")
    s = lax.axis_index("s")
    w = _wid(c, s)
    iota = lax.iota(i32, 16)

    pltpu.sync_copy(recip_hbm, recip_v)

    def chunk(ci, _):
        ebase = w * PER_TILE + ci * KA
        pltpu.sync_copy(dstp.at[pl.ds(ebase, KA)], didx)
        pltpu.sync_copy(ex_hbm.at[pl.ds(ebase * H, KA * H)], exb)
        for g in range(KA // 16):
            dv = didx[pl.ds(g * 16, 16)]
            pos = (g * 16 + iota) * H
            for h in range(H):
                rv = plsc.load_gather(recip_v, [dv * H + h])
                exv = plsc.load_gather(exb, [pos + h])
                plsc.store_scatter(attnb, [pos + h], rv * exv)
        pltpu.sync_copy(attnb, attn_hbm.at[pl.ds(ebase * H, KA * H)])
        return _

    lax.fori_loop(0, NCHA, chunk, None)


def _attn_scale(recip_flat, ex, dstp):
    return pl.kernel(
        _attn_scale_body,
        out_type=jax.ShapeDtypeStruct((EPAD * H,), f32),
        mesh=_MESH,
        compiler_params=pltpu.CompilerParams(needs_layout_passes=False),
        scratch_types=[
            pltpu.VMEM((NPAD * H,), f32),
            pltpu.VMEM((KA,), i32),
            pltpu.VMEM((KA * H,), f32),
            pltpu.VMEM((KA * H,), f32),
        ],
    )(recip_flat, ex, dstp)


def _hop_body(feat_hbm, attn_hbm, srcp, dstp, zeros128, apart_hbm,
              rows0, sidx0, didx0, scaleb0, rows1, sidx1, didx1, scaleb1,
              agg_sh, g0, g1):
    c = lax.axis_index("c")
    s = lax.axis_index("s")
    w = _wid(c, s)

    pltpu.sync_copy(zeros128.at[pl.ds(s * ROWS_PER_TILE, ROWS_PER_TILE)],
                    agg_sh.at[pl.ds(s * ROWS_PER_TILE, ROWS_PER_TILE)])
    plsc.subcore_barrier()

    def issue(ci, sidx, didx, scaleb, rows, gsem):
        ebase = w * PER_TILE + ci * K
        pltpu.sync_copy(srcp.at[pl.ds(ebase, K)], sidx)
        pltpu.sync_copy(dstp.at[pl.ds(ebase, K)], didx)
        pltpu.sync_copy(attn_hbm.at[pl.ds(ebase * H, K * H)], scaleb)
        pltpu.async_copy(feat_hbm.at[sidx], rows, gsem)

    def process(sidx, didx, scaleb, rows, gsem):
        pltpu.make_async_copy(feat_hbm.at[sidx], rows, gsem).wait()

        def edge4(e4, _):
            scv16 = scaleb[pl.ds(e4 * 16, 16)]
            for q in range(4):
                e = e4 * 4 + q
                for h in range(H):
                    lane = jnp.full((16,), q * H + h, i32)
                    scv = scv16.at[lane].get(mode="promise_in_bounds")
                    for j in range(DH // 16):
                        off = h * DH + j * 16
                        rows[e, pl.ds(off, 16)] = rows[e, pl.ds(off, 16)] * scv
            return _

        lax.fori_loop(0, K // 4, edge4, None)
        pltpu.sync_copy(rows, agg_sh.at[didx], add=True)

    NPAIR = NCHUNK // 2
    issue(0, sidx0, didx0, scaleb0, rows0, g0)

    def pair(cj, _):
        issue(2 * cj + 1, sidx1, didx1, scaleb1, rows1, g1)
        process(sidx0, didx0, scaleb0, rows0, g0)

        @pl.when(cj < NPAIR - 1)
        def _issue_next():
            issue(2 * cj + 2, sidx0, didx0, scaleb0, rows0, g0)

        process(sidx1, didx1, scaleb1, rows1, g1)
        return _

    lax.fori_loop(0, NPAIR, pair, None)
    plsc.subcore_barrier()
    r0 = s * ROWS_PER_TILE
    pltpu.sync_copy(agg_sh.at[pl.ds(r0, ROWS_PER_TILE)],
                    apart_hbm.at[pl.ds(c * NPAD + r0, ROWS_PER_TILE)])


def _hop(feat, attn, srcp, dstp, zeros128):
    return pl.kernel(
        _hop_body,
        out_type=jax.ShapeDtypeStruct((2 * NPAD, D), f32),
        mesh=_MESH,
        compiler_params=pltpu.CompilerParams(needs_layout_passes=False),
        scratch_types=[
            pltpu.VMEM((K, D), f32),
            pltpu.VMEM((K,), i32),
            pltpu.VMEM((K,), i32),
            pltpu.VMEM((K * H,), f32),
            pltpu.VMEM((K, D), f32),
            pltpu.VMEM((K,), i32),
            pltpu.VMEM((K,), i32),
            pltpu.VMEM((K * H,), f32),
            pltpu.VMEM_SHARED((NPAD, D), f32),
            pltpu.SemaphoreType.DMA,
            pltpu.SemaphoreType.DMA,
        ],
    )(feat, attn, srcp, dstp, zeros128)


def _layer(h_in, w, p, srcp, dstp, zeros4, zeros128):
    hp, elr = _proj(h_in, w, p)
    ex, dparts = _attn(elr.reshape(-1), srcp, dstp, zeros4)
    recip = _recip(dparts.reshape(NTILES, NPAD * H)).reshape(-1)
    attn = _attn_scale(recip, ex, dstp)
    feat = hp
    parts = None
    for k in range(HOPS):
        parts = _hop(feat, attn, srcp, dstp, zeros128)
        if k < HOPS - 1:
            feat = _combine_mid(parts, hp)
    return parts, hp


def kernel(x, edge_index, W1, al1, ar1, W2, al2, ar2, Wc, bc):
    xp = jnp.zeros((NPAD, D), f32).at[:N].set(x)
    srcp = jnp.zeros((EPAD,), i32).at[:E].set(edge_index[0])
    dstp = jnp.zeros((EPAD,), i32).at[:E].set(edge_index[1])
    mask = jnp.repeat(jnp.eye(H, dtype=f32), DH, axis=0)
    p1 = jnp.concatenate([mask * al1.reshape(-1)[:, None],
                          mask * ar1.reshape(-1)[:, None]], axis=1)
    p2 = jnp.concatenate([mask * al2.reshape(-1)[:, None],
                          mask * ar2.reshape(-1)[:, None]], axis=1)
    wc_pad = jnp.zeros((D, D), f32).at[:, :C].set(Wc)
    bc_pad = jnp.zeros((1, D), f32).at[0, :C].set(bc)
    zeros4 = jnp.zeros((NPAD * H,), f32)
    zeros128 = jnp.zeros((NPAD, D), f32)

    parts1, hp1 = _layer(xp, W1, p1, srcp, dstp, zeros4, zeros128)
    h1 = _combine_end(parts1, hp1, xp)
    parts2, hp2 = _layer(h1, W2, p2, srcp, dstp, zeros4, zeros128)
    logits_pad = _classify(parts2, hp2, h1, wc_pad, bc_pad)
    return logits_pad[:N, :C]
